# baseline (speedup 1.0000x reference)
import functools

import jax
import jax.numpy as jnp
from jax import lax
from jax.experimental import pallas as pl
from jax.experimental.pallas import tpu as pltpu
from jax.experimental.pallas import tpu_sc as plsc

N = 10000
E = 320000
F_IN = 128
HID = 128
NCLS = 64

NC = 2
NS = 16
LANES = 16

NPAD = 10240
TRASHN = NPAD - N
E2 = 327680
EPT = E2 // NS
ECH = 128
NCHUNK = EPT // ECH
GB = 2
NPT = NPAD // NS
NODE_CH = 40
NNCH = NPT // NODE_CH

_mesh = plsc.VectorSubcoreMesh(core_axis_name="c", subcore_axis_name="s")


def _f32(x):
    return jnp.asarray(x, jnp.float32)


def _make_prop(W, fuse_deg):
    QN = W // LANES
    CHW = 8192 // W
    NGW = EPT // CHW
    GROUPS = NGW // GB

    outs = (
        jax.ShapeDtypeStruct((NC * NPAD, W), jnp.float32),
        jax.ShapeDtypeStruct((NC * NPAD, W), jnp.float32),
    )
    if fuse_deg:
        outs = outs + (
            jax.ShapeDtypeStruct((NPAD,), jnp.float32),
            jax.ShapeDtypeStruct((NPAD,), jnp.float32),
            jax.ShapeDtypeStruct((E2 // 128, 128), jnp.int32),
        )

    scratch = dict(
        rixb=pltpu.VMEM((4, GB, CHW), jnp.int32),
        cixb=pltpu.VMEM((4, GB, CHW), jnp.int32),
        gbuf=pltpu.VMEM((2 * GB, CHW, W), jnp.float32),
        nsy=pltpu.VMEM((NODE_CH, W), jnp.float32),
        nu=pltpu.VMEM((NODE_CH, W), jnp.float32),
        zbuf=pltpu.VMEM((NODE_CH, W), jnp.float32),
        s2b=pltpu.VMEM((NPT,), jnp.float32),
        sb=pltpu.VMEM((NPT,), jnp.float32),
        acc_sh=pltpu.VMEM_SHARED((NPAD, W), jnp.float32),
        gsem=pltpu.SemaphoreType.DMA((2 * GB,)),
        ssem=pltpu.SemaphoreType.DMA((2 * GB,)),
        risem=pltpu.SemaphoreType.DMA((4,)),
        cisem=pltpu.SemaphoreType.DMA((4,)),
        asem=pltpu.SemaphoreType.DMA,
        bsem=pltpu.SemaphoreType.DMA,
    )
    if fuse_deg:
        scratch.update(
            rbuf=pltpu.VMEM((8, 128), jnp.int32),
            cbuf=pltpu.VMEM((8, 128), jnp.int32),
            wbuf=pltpu.VMEM((8, 128), jnp.float32),
            degacc=pltpu.VMEM_SHARED((NPAD,), jnp.float32),
        )

    @functools.partial(
        pl.kernel,
        out_type=outs,
        mesh=_mesh,
        scratch_types=scratch,
        compiler_params=pltpu.CompilerParams(needs_layout_passes=False,
                                             use_tc_tiling_on_sc=False),
    )
    def prop(rowoff_hbm, colx_hbm, u3_hbm, sy2_hbm, sy1_hbm, *rest,
             rixb, cixb, gbuf, nsy, nu, zbuf, s2b, sb, acc_sh, gsem, ssem,
             risem, cisem, asem, bsem, **xscr):
        if fuse_deg:
            m_hbm, uw_hbm, sv_hbm, s2v_hbm, colp_hbm = rest
            rbuf, cbuf, wbuf = xscr["rbuf"], xscr["cbuf"], xscr["wbuf"]
            degacc = xscr["degacc"]
        else:
            s2_hbm, m_hbm, uw_hbm = rest
            colp_hbm = colx_hbm
        c = lax.axis_index("c")
        s = lax.axis_index("s")
        nbase = s * NPT
        ebase = s * NGW
        row_hbm = rowoff_hbm.at[c]

        if fuse_deg:
            def _zs(t, _):
                sb[pl.ds(t * LANES, LANES)] = jnp.zeros((LANES,),
                                                        jnp.float32)
                return _
            lax.fori_loop(0, NPT // LANES, _zs, None)
            pltpu.sync_copy(sb, degacc.at[pl.ds(nbase, NPT)])
            plsc.subcore_barrier()

            erow = s * (EPT // 128)
            raw_rows = rowoff_hbm.at[0]

            def echunk(ch, _):
                base = erow + ch * 8
                pltpu.sync_copy(raw_rows.at[pl.ds(base, 8)], rbuf)
                pltpu.sync_copy(colx_hbm.at[pl.ds(base, 8)], cbuf)

                def vec(t, _2):
                    j = t // 8
                    k = t % 8
                    sl = pl.ds(k * LANES, LANES)
                    r = rbuf[j, sl]
                    cc = cbuf[j, sl]
                    msk = r != cc
                    cbuf[j, sl] = jnp.where(
                        msk, cc, N + jnp.remainder(cc, TRASHN))
                    wbuf[j, sl] = jnp.where(msk, 1.0, 0.0)
                    return _2
                lax.fori_loop(0, 64, vec, None)

                pltpu.sync_copy(cbuf, colp_hbm.at[pl.ds(base, 8)])
                for j in range(8):
                    pltpu.async_copy(wbuf.at[j], degacc.at[rbuf.at[j]],
                                     asem, add=True)
                for j in range(8):
                    pltpu.make_async_copy(wbuf.at[j], degacc.at[rbuf.at[j]],
                                          asem).wait()
                return _
            lax.fori_loop(0, EPT // 1024, echunk, None)
            plsc.subcore_barrier()

            pltpu.sync_copy(degacc.at[pl.ds(nbase, NPT)], s2b)

            def _deg(t, _):
                sl = pl.ds(t * LANES, LANES)
                d = s2b[sl]
                pos = d > 0.0
                i = plsc.bitcast(d, jnp.int32)
                i = jnp.full((LANES,), 0x5F3759DF, jnp.int32) - \
                    lax.shift_right_logical(i, jnp.full((LANES,), 1,
                                                        jnp.int32))
                y = plsc.bitcast(i, jnp.float32)
                for _it in range(3):
                    y = y * (1.5 - 0.5 * d * y * y)
                s2b[sl] = jnp.where(pos, 1.0 / d, 0.0)
                sb[sl] = jnp.where(pos, y, 0.0)
                return _
            lax.fori_loop(0, NPT // LANES, _deg, None)

            @pl.when(c == 0)
            def _():
                pltpu.sync_copy(sb, sv_hbm.at[pl.ds(nbase, NPT)])
                pltpu.sync_copy(s2b, s2v_hbm.at[pl.ds(nbase, NPT)])
        else:
            pltpu.sync_copy(s2_hbm.at[pl.ds(nbase, NPT)], s2b)

        def _z(t, _):
            def _zrow(q, _2):
                zbuf[t, pl.ds(q * LANES, LANES)] = jnp.zeros((LANES,),
                                                             jnp.float32)
                return _2
            lax.fori_loop(0, QN, _zrow, None)
            return _
        lax.fori_loop(0, NODE_CH, _z, None)

        for q in range(NNCH):
            r0 = nbase + q * NODE_CH
            pltpu.sync_copy(zbuf, acc_sh.at[pl.ds(r0, NODE_CH)])
            if fuse_deg:
                pltpu.sync_copy(u3_hbm.at[pl.ds(c * NPAD + r0, NODE_CH)],
                                nsy)

                def _su3(j, _):
                    jj = q * NODE_CH + j
                    sv = plsc.load_gather(
                        sb, [jnp.full((LANES,), jj, jnp.int32)])
                    for qq in range(QN):
                        sl = pl.ds(qq * LANES, LANES)
                        nu[j, sl] = sv * nsy[j, sl]
                    return _
                lax.fori_loop(0, NODE_CH, _su3, None)
                pltpu.sync_copy(nu, uw_hbm.at[pl.ds(c * NPAD + r0, NODE_CH)])
        plsc.subcore_barrier()

        def idx_load(g, p):
            pltpu.async_copy(row_hbm.at[pl.ds(ebase + g * GB, GB)],
                             rixb.at[p], risem.at[p])
            pltpu.async_copy(colp_hbm.at[pl.ds(ebase + g * GB, GB)],
                             cixb.at[p], cisem.at[p])

        def sweep(u_hbm):
            idx_load(0, 0)
            idx_load(1, 1)

            def do_group(g, k):
                p2 = k % 2
                po = (k + 2) % 4
                rix = rixb.at[k]
                cix = cixb.at[k]
                pltpu.make_async_copy(
                    row_hbm.at[pl.ds(ebase + g * GB, GB)], rix,
                    risem.at[k]).wait()

                @pl.when(g >= 2)
                def _():
                    for b in range(GB):
                        pltpu.make_async_copy(
                            gbuf.at[p2 * GB + b],
                            acc_sh.at[cixb.at[po].at[b]],
                            ssem.at[p2 * GB + b]).wait()

                @pl.when(g + 2 < GROUPS)
                def _():
                    idx_load(g + 2, po)

                pltpu.make_async_copy(
                    colp_hbm.at[pl.ds(ebase + g * GB, GB)], cix,
                    cisem.at[k]).wait()
                for b in range(GB):
                    pltpu.async_copy(u_hbm.at[rix.at[b]],
                                     gbuf.at[p2 * GB + b],
                                     gsem.at[p2 * GB + b])
                for b in range(GB):
                    pltpu.make_async_copy(u_hbm.at[rix.at[b]],
                                          gbuf.at[p2 * GB + b],
                                          gsem.at[p2 * GB + b]).wait()
                    pltpu.async_copy(gbuf.at[p2 * GB + b],
                                     acc_sh.at[cix.at[b]],
                                     ssem.at[p2 * GB + b], add=True)

            def quad(it, _):
                for k in range(4):
                    do_group(4 * it + k, k)
                return _
            lax.fori_loop(0, GROUPS // 4, quad, None)
            for k in (2, 3):
                p2 = k % 2
                for b in range(GB):
                    pltpu.make_async_copy(
                        gbuf.at[p2 * GB + b],
                        acc_sh.at[cixb.at[k].at[b]],
                        ssem.at[p2 * GB + b]).wait()
            plsc.subcore_barrier()

        def rescale(sy_hbm):
            for q in range(NNCH):
                r0 = nbase + q * NODE_CH
                pltpu.async_copy(acc_sh.at[pl.ds(r0, NODE_CH)], nu, asem)
                pltpu.async_copy(sy_hbm.at[pl.ds(c * NPAD + r0, NODE_CH)],
                                 nsy, bsem)
                pltpu.make_async_copy(acc_sh.at[pl.ds(r0, NODE_CH)], nu,
                                      asem).wait()
                pltpu.make_async_copy(sy_hbm.at[pl.ds(c * NPAD + r0, NODE_CH)],
                                      nsy, bsem).wait()

                def node(j, _):
                    jj = q * NODE_CH + j
                    idxv = jnp.full((LANES,), jj, jnp.int32)
                    s2v = plsc.load_gather(s2b, [idxv])
                    if fuse_deg:
                        sv = plsc.load_gather(sb, [idxv])
                    for qq in range(QN):
                        sl = pl.ds(qq * LANES, LANES)
                        if fuse_deg:
                            nu[j, sl] = sv * nsy[j, sl] + s2v * nu[j, sl]
                        else:
                            nu[j, sl] = nsy[j, sl] + s2v * nu[j, sl]
                    return _
                lax.fori_loop(0, NODE_CH, node, None)

                pltpu.sync_copy(nu, uw_hbm.at[pl.ds(c * NPAD + r0, NODE_CH)])
                pltpu.sync_copy(zbuf, acc_sh.at[pl.ds(r0, NODE_CH)])
            plsc.subcore_barrier()

        sweep(uw_hbm if fuse_deg else u3_hbm)
        rescale(sy2_hbm)
        sweep(uw_hbm)
        rescale(sy1_hbm)
        sweep(uw_hbm)

        for q in range(NNCH):
            r0 = nbase + q * NODE_CH
            pltpu.sync_copy(acc_sh.at[pl.ds(r0, NODE_CH)], nu)
            pltpu.sync_copy(nu, m_hbm.at[pl.ds(c * NPAD + r0, NODE_CH)])

    return prop


_prop64 = _make_prop(64, fuse_deg=True)
_prop32 = _make_prop(32, fuse_deg=False)


_BN = 128
_GRID = NPAD // _BN


def _tc_prep_body(x_ref, w_ref, y0_ref, y1_ref, y2_ref, y3_ref):
    y = jnp.dot(x_ref[...], w_ref[...], preferred_element_type=jnp.float32)
    y0_ref[...] = y[:, :HID]
    y1_ref[...] = y[:, HID:2 * HID]
    y2_ref[...] = y[:, 2 * HID:3 * HID]
    y3_ref[...] = y[:, 3 * HID:]


def _tc_prep(xp, wcat):
    f32 = jnp.float32
    outs = tuple(jax.ShapeDtypeStruct((NPAD, HID), f32) for _ in range(4))
    blk = pl.BlockSpec((_BN, HID), lambda i: (i, 0))
    return pl.pallas_call(
        _tc_prep_body,
        grid=(_GRID,),
        in_specs=[
            pl.BlockSpec((_BN, F_IN), lambda i: (i, 0)),
            pl.BlockSpec((F_IN, 4 * HID), lambda i: (0, 0)),
        ],
        out_specs=[blk, blk, blk, blk],
        out_shape=outs,
    )(xp, wcat)


def _tc_mid_body(m1_ref, y0_ref, s_ref, b1_ref, w_ref, z0_ref, u3_ref,
                 sz2_ref, sz1_ref):
    sv = s_ref[...]
    h = jnp.maximum(y0_ref[...] + sv * m1_ref[...] + b1_ref[...], 0.0)
    z = jnp.dot(h, w_ref[...], preferred_element_type=jnp.float32)
    z0_ref[...] = z[:, :NCLS]
    sz1_ref[...] = sv * z[:, NCLS:2 * NCLS]
    sz2_ref[...] = sv * z[:, 2 * NCLS:3 * NCLS]
    u3_ref[...] = sv * z[:, 3 * NCLS:]


def _tc_mid(m1, y0, svec, b1, wcat):
    f32 = jnp.float32
    outs = tuple(jax.ShapeDtypeStruct((NPAD, NCLS), f32) for _ in range(4))
    blk = pl.BlockSpec((_BN, NCLS), lambda i: (i, 0))
    return pl.pallas_call(
        _tc_mid_body,
        grid=(_GRID,),
        in_specs=[
            pl.BlockSpec((_BN, HID), lambda i: (i, 0)),
            pl.BlockSpec((_BN, HID), lambda i: (i, 0)),
            pl.BlockSpec((_BN, 1), lambda i: (i, 0)),
            pl.BlockSpec((1, HID), lambda i: (0, 0)),
            pl.BlockSpec((HID, 4 * NCLS), lambda i: (0, 0)),
        ],
        out_specs=[blk, blk, blk, blk],
        out_shape=outs,
    )(m1, y0, svec, b1, wcat)


def _tc_post_body(z0_ref, m2_ref, s_ref, b2_ref, out_ref):
    o = z0_ref[...] + s_ref[...] * m2_ref[...] + b2_ref[...]
    mx = jnp.max(o, axis=1, keepdims=True)
    ex = jnp.exp(o - mx)
    lse = mx + jnp.log(jnp.sum(ex, axis=1, keepdims=True))
    out_ref[...] = o - lse


def _tc_post(z0, m2, svec, b2):
    return pl.pallas_call(
        _tc_post_body,
        grid=(_GRID,),
        in_specs=[
            pl.BlockSpec((_BN, NCLS), lambda i: (i, 0)),
            pl.BlockSpec((_BN, NCLS), lambda i: (i, 0)),
            pl.BlockSpec((_BN, 1), lambda i: (i, 0)),
            pl.BlockSpec((1, NCLS), lambda i: (0, 0)),
        ],
        out_specs=pl.BlockSpec((_BN, NCLS), lambda i: (i, 0)),
        out_shape=jax.ShapeDtypeStruct((NPAD, NCLS), jnp.float32),
    )(z0, m2, svec, b2)


def _split_cols(a, w):
    return a.reshape(NPAD, 2, w).transpose(1, 0, 2).reshape(2 * NPAD, w)


def _merge_cols(a, w):
    return a.reshape(2, NPAD, w).transpose(1, 0, 2).reshape(NPAD, 2 * w)


def _fold(W):
    return jnp.concatenate(
        [W[0] - W[2], 3.0 * W[3] - W[1], 2.0 * W[2], -4.0 * W[3]], axis=1)


def kernel(x, edge_index, W1, b1, W2, b2):
    x = _f32(x)
    wcat1 = _fold(_f32(W1))
    wcat2 = _fold(_f32(W2))

    row = edge_index[0].astype(jnp.int32)
    col = edge_index[1].astype(jnp.int32)
    padv = (jnp.arange(E, E2, dtype=jnp.int32)) % N
    rowp = jnp.concatenate([row, padv]).reshape(E2 // 128, 128)
    colp_in = jnp.concatenate([col, padv]).reshape(E2 // 128, 128)
    rowoff = jnp.stack([rowp, rowp + NPAD])

    rowoff256 = rowoff.reshape(NC, E2 // 256, 256)

    xp = jnp.pad(x, ((0, NPAD - N), (0, 0)))
    y0, y1, y2, y3 = _tc_prep(xp, wcat1)

    m1s, _, svec, s2flat, colp = _prop64(rowoff, colp_in,
                                         _split_cols(y3, 64),
                                         _split_cols(y2, 64),
                                         _split_cols(y1, 64))
    colp256 = colp.reshape(E2 // 256, 256)
    m1 = _merge_cols(m1s, 64)
    svec2 = svec.reshape(NPAD, 1)

    z0, u3z, sz2, sz1 = _tc_mid(m1, y0, svec2, b1.reshape(1, HID), wcat2)

    m2s, _ = _prop32(rowoff256, colp256, _split_cols(u3z, 32),
                     _split_cols(sz2, 32), _split_cols(sz1, 32), s2flat)
    m2 = _merge_cols(m2s, 32)

    out = _tc_post(z0, m2, svec2, b2.reshape(1, NCLS))
    return out[:N]


# speedup vs baseline: 1.0857x; 1.0857x over previous
import functools

import jax
import jax.numpy as jnp
from jax import lax
from jax.experimental import pallas as pl
from jax.experimental.pallas import tpu as pltpu
from jax.experimental.pallas import tpu_sc as plsc

N = 10000
E = 320000
F_IN = 128
HID = 128
NCLS = 64

NC = 2
NS = 16
LANES = 16

NPAD = 10240
TRASHN = NPAD - N
E2 = 327680
EPT = E2 // NS
ECH = 128
NCHUNK = EPT // ECH
GB = 2
NPT = NPAD // NS
NODE_CH = 40
NNCH = NPT // NODE_CH

EPW = E2 // (NC * NS)
PCH = 2048
PROWS = PCH // 128

_mesh = plsc.VectorSubcoreMesh(core_axis_name="c", subcore_axis_name="s")


def _f32(x):
    return jnp.asarray(x, jnp.float32)


@functools.partial(
    pl.kernel,
    out_type=(
        jax.ShapeDtypeStruct((NC * NPAD,), jnp.float32),
        jax.ShapeDtypeStruct((E2 // 128, 128), jnp.int32),
    ),
    mesh=_mesh,
    scratch_types=dict(
        rbuf=pltpu.VMEM((PROWS, 128), jnp.int32),
        cbuf=pltpu.VMEM((PROWS, 128), jnp.int32),
        cpbuf=pltpu.VMEM((PROWS, 128), jnp.int32),
        wbuf=pltpu.VMEM((PROWS, 128), jnp.float32),
        zbuf=pltpu.VMEM((NPT,), jnp.float32),
        degacc=pltpu.VMEM_SHARED((NPAD,), jnp.float32),
        sem=pltpu.SemaphoreType.DMA,
    ),
)
def _edge_prep(row_hbm, col_hbm, deg_hbm, colp_hbm, rbuf, cbuf, cpbuf, wbuf,
               zbuf, degacc, sem):
    c = lax.axis_index("c")
    s = lax.axis_index("s")
    wid = c * NS + s

    def _z(t, _):
        zbuf[pl.ds(t * LANES, LANES)] = jnp.zeros((LANES,), jnp.float32)
        return _
    lax.fori_loop(0, NPT // LANES, _z, None)
    pltpu.sync_copy(zbuf, degacc.at[pl.ds(s * NPT, NPT)])
    plsc.subcore_barrier()

    def chunk(ch, _):
        base = wid * (EPW // 128) + ch * PROWS
        pltpu.sync_copy(row_hbm.at[pl.ds(base, PROWS)], rbuf)
        pltpu.sync_copy(col_hbm.at[pl.ds(base, PROWS)], cbuf)

        def vec(t, _):
            j = t // (128 // LANES)
            k = t % (128 // LANES)
            r = rbuf[j, pl.ds(k * LANES, LANES)]
            cc = cbuf[j, pl.ds(k * LANES, LANES)]
            m = r != cc
            cpbuf[j, pl.ds(k * LANES, LANES)] = jnp.where(
                m, cc, N + jnp.remainder(cc, TRASHN))
            wbuf[j, pl.ds(k * LANES, LANES)] = jnp.where(m, 1.0, 0.0)
            return _
        lax.fori_loop(0, PROWS * (128 // LANES), vec, None)

        pltpu.sync_copy(cpbuf, colp_hbm.at[pl.ds(base, PROWS)])
        for j in range(PROWS):
            pltpu.sync_copy(wbuf.at[j], degacc.at[rbuf.at[j]], add=True)
        return _
    lax.fori_loop(0, EPW // PCH, chunk, None)
    plsc.subcore_barrier()

    pltpu.sync_copy(degacc.at[pl.ds(s * NPT, NPT)], zbuf)
    pltpu.sync_copy(zbuf, deg_hbm.at[pl.ds(c * NPAD + s * NPT, NPT)])


def _make_prop(W, fuse_deg):
    QN = W // LANES
    CHW = 8192 // W
    NGW = EPT // CHW
    GROUPS = NGW // GB

    outs = (
        jax.ShapeDtypeStruct((NC * NPAD, W), jnp.float32),
        jax.ShapeDtypeStruct((NC * NPAD, W), jnp.float32),
    )
    if fuse_deg:
        outs = outs + (
            jax.ShapeDtypeStruct((NPAD,), jnp.float32),
            jax.ShapeDtypeStruct((NPAD,), jnp.float32),
        )

    scratch = dict(
        rixb=pltpu.VMEM((4, GB, CHW), jnp.int32),
        cixb=pltpu.VMEM((4, GB, CHW), jnp.int32),
        gbuf=pltpu.VMEM((2 * GB, CHW, W), jnp.float32),
        nsy=pltpu.VMEM((NODE_CH, W), jnp.float32),
        nu=pltpu.VMEM((NODE_CH, W), jnp.float32),
        zbuf=pltpu.VMEM((NODE_CH, W), jnp.float32),
        s2b=pltpu.VMEM((NPT,), jnp.float32),
        sb=pltpu.VMEM((NPT,), jnp.float32),
        acc_sh=pltpu.VMEM_SHARED((NPAD, W), jnp.float32),
        gsem=pltpu.SemaphoreType.DMA((2 * GB,)),
        ssem=pltpu.SemaphoreType.DMA((2 * GB,)),
        risem=pltpu.SemaphoreType.DMA((4,)),
        cisem=pltpu.SemaphoreType.DMA((4,)),
        asem=pltpu.SemaphoreType.DMA,
        bsem=pltpu.SemaphoreType.DMA,
    )
    @functools.partial(
        pl.kernel,
        out_type=outs,
        mesh=_mesh,
        scratch_types=scratch,
        compiler_params=pltpu.CompilerParams(needs_layout_passes=False,
                                             use_tc_tiling_on_sc=False),
    )
    def prop(rowoff_hbm, colx_hbm, u3_hbm, sy2_hbm, sy1_hbm, *rest,
             rixb, cixb, gbuf, nsy, nu, zbuf, s2b, sb, acc_sh, gsem, ssem,
             risem, cisem, asem, bsem, **xscr):
        del xscr
        if fuse_deg:
            deg_hbm, m_hbm, uw_hbm, sv_hbm, s2v_hbm = rest
        else:
            s2_hbm, m_hbm, uw_hbm = rest
        colp_hbm = colx_hbm
        c = lax.axis_index("c")
        s = lax.axis_index("s")
        nbase = s * NPT
        ebase = s * NGW
        row_hbm = rowoff_hbm.at[c]

        if fuse_deg:
            pltpu.async_copy(deg_hbm.at[pl.ds(nbase, NPT)], s2b, asem)
            pltpu.async_copy(deg_hbm.at[pl.ds(NPAD + nbase, NPT)], sb, bsem)
            pltpu.make_async_copy(deg_hbm.at[pl.ds(nbase, NPT)], s2b,
                                  asem).wait()
            pltpu.make_async_copy(deg_hbm.at[pl.ds(NPAD + nbase, NPT)], sb,
                                  bsem).wait()

            def _deg(t, _):
                sl = pl.ds(t * LANES, LANES)
                d = s2b[sl] + sb[sl]
                pos = d > 0.0
                i = plsc.bitcast(d, jnp.int32)
                i = jnp.full((LANES,), 0x5F3759DF, jnp.int32) - \
                    lax.shift_right_logical(i, jnp.full((LANES,), 1,
                                                        jnp.int32))
                y = plsc.bitcast(i, jnp.float32)
                for _it in range(3):
                    y = y * (1.5 - 0.5 * d * y * y)
                s2b[sl] = jnp.where(pos, 1.0 / d, 0.0)
                sb[sl] = jnp.where(pos, y, 0.0)
                return _
            lax.fori_loop(0, NPT // LANES, _deg, None)

            @pl.when(c == 0)
            def _():
                pltpu.sync_copy(sb, sv_hbm.at[pl.ds(nbase, NPT)])
                pltpu.sync_copy(s2b, s2v_hbm.at[pl.ds(nbase, NPT)])
        else:
            pltpu.sync_copy(s2_hbm.at[pl.ds(nbase, NPT)], s2b)

        def _z(t, _):
            def _zrow(q, _2):
                zbuf[t, pl.ds(q * LANES, LANES)] = jnp.zeros((LANES,),
                                                             jnp.float32)
                return _2
            lax.fori_loop(0, QN, _zrow, None)
            return _
        lax.fori_loop(0, NODE_CH, _z, None)

        for q in range(NNCH):
            r0 = nbase + q * NODE_CH
            pltpu.sync_copy(zbuf, acc_sh.at[pl.ds(r0, NODE_CH)])
            if fuse_deg:
                pltpu.sync_copy(u3_hbm.at[pl.ds(c * NPAD + r0, NODE_CH)],
                                nsy)

                def _su3(j, _):
                    jj = q * NODE_CH + j
                    sv = plsc.load_gather(
                        sb, [jnp.full((LANES,), jj, jnp.int32)])
                    for qq in range(QN):
                        sl = pl.ds(qq * LANES, LANES)
                        nu[j, sl] = sv * nsy[j, sl]
                    return _
                lax.fori_loop(0, NODE_CH, _su3, None)
                pltpu.sync_copy(nu, uw_hbm.at[pl.ds(c * NPAD + r0, NODE_CH)])
        plsc.subcore_barrier()

        def idx_load(g, p):
            pltpu.async_copy(row_hbm.at[pl.ds(ebase + g * GB, GB)],
                             rixb.at[p], risem.at[p])
            pltpu.async_copy(colp_hbm.at[pl.ds(ebase + g * GB, GB)],
                             cixb.at[p], cisem.at[p])

        def sweep(u_hbm):
            idx_load(0, 0)
            idx_load(1, 1)

            def do_group(g, k):
                p2 = k % 2
                po = (k + 2) % 4
                rix = rixb.at[k]
                cix = cixb.at[k]
                pltpu.make_async_copy(
                    row_hbm.at[pl.ds(ebase + g * GB, GB)], rix,
                    risem.at[k]).wait()

                @pl.when(g >= 2)
                def _():
                    for b in range(GB):
                        pltpu.make_async_copy(
                            gbuf.at[p2 * GB + b],
                            acc_sh.at[cixb.at[po].at[b]],
                            ssem.at[p2 * GB + b]).wait()

                @pl.when(g + 2 < GROUPS)
                def _():
                    idx_load(g + 2, po)

                pltpu.make_async_copy(
                    colp_hbm.at[pl.ds(ebase + g * GB, GB)], cix,
                    cisem.at[k]).wait()
                for b in range(GB):
                    pltpu.async_copy(u_hbm.at[rix.at[b]],
                                     gbuf.at[p2 * GB + b],
                                     gsem.at[p2 * GB + b])
                for b in range(GB):
                    pltpu.make_async_copy(u_hbm.at[rix.at[b]],
                                          gbuf.at[p2 * GB + b],
                                          gsem.at[p2 * GB + b]).wait()
                    pltpu.async_copy(gbuf.at[p2 * GB + b],
                                     acc_sh.at[cix.at[b]],
                                     ssem.at[p2 * GB + b], add=True)

            def quad(it, _):
                for k in range(4):
                    do_group(4 * it + k, k)
                return _
            lax.fori_loop(0, GROUPS // 4, quad, None)
            for k in (2, 3):
                p2 = k % 2
                for b in range(GB):
                    pltpu.make_async_copy(
                        gbuf.at[p2 * GB + b],
                        acc_sh.at[cixb.at[k].at[b]],
                        ssem.at[p2 * GB + b]).wait()
            plsc.subcore_barrier()

        def rescale(sy_hbm):
            for q in range(NNCH):
                r0 = nbase + q * NODE_CH
                pltpu.async_copy(acc_sh.at[pl.ds(r0, NODE_CH)], nu, asem)
                pltpu.async_copy(sy_hbm.at[pl.ds(c * NPAD + r0, NODE_CH)],
                                 nsy, bsem)
                pltpu.make_async_copy(acc_sh.at[pl.ds(r0, NODE_CH)], nu,
                                      asem).wait()
                pltpu.make_async_copy(sy_hbm.at[pl.ds(c * NPAD + r0, NODE_CH)],
                                      nsy, bsem).wait()

                def node(j, _):
                    jj = q * NODE_CH + j
                    idxv = jnp.full((LANES,), jj, jnp.int32)
                    s2v = plsc.load_gather(s2b, [idxv])
                    if fuse_deg:
                        sv = plsc.load_gather(sb, [idxv])
                    for qq in range(QN):
                        sl = pl.ds(qq * LANES, LANES)
                        if fuse_deg:
                            nu[j, sl] = sv * nsy[j, sl] + s2v * nu[j, sl]
                        else:
                            nu[j, sl] = nsy[j, sl] + s2v * nu[j, sl]
                    return _
                lax.fori_loop(0, NODE_CH, node, None)

                pltpu.sync_copy(nu, uw_hbm.at[pl.ds(c * NPAD + r0, NODE_CH)])
                pltpu.sync_copy(zbuf, acc_sh.at[pl.ds(r0, NODE_CH)])
            plsc.subcore_barrier()

        sweep(uw_hbm if fuse_deg else u3_hbm)
        rescale(sy2_hbm)
        sweep(uw_hbm)
        rescale(sy1_hbm)
        sweep(uw_hbm)

        for q in range(NNCH):
            r0 = nbase + q * NODE_CH
            pltpu.sync_copy(acc_sh.at[pl.ds(r0, NODE_CH)], nu)
            pltpu.sync_copy(nu, m_hbm.at[pl.ds(c * NPAD + r0, NODE_CH)])

    return prop


_prop64 = _make_prop(64, fuse_deg=True)
_prop32 = _make_prop(32, fuse_deg=False)


_BN = 128
_GRID = NPAD // _BN


def _tc_prep_body(x_ref, w_ref, y0_ref, y1_ref, y2_ref, y3_ref):
    y = jnp.dot(x_ref[...], w_ref[...], preferred_element_type=jnp.float32)
    y0_ref[...] = y[:, :HID]
    y1_ref[...] = y[:, HID:2 * HID]
    y2_ref[...] = y[:, 2 * HID:3 * HID]
    y3_ref[...] = y[:, 3 * HID:]


def _tc_prep(xp, wcat):
    f32 = jnp.float32
    outs = tuple(jax.ShapeDtypeStruct((NPAD, HID), f32) for _ in range(4))
    blk = pl.BlockSpec((_BN, HID), lambda i: (i, 0))
    return pl.pallas_call(
        _tc_prep_body,
        grid=(_GRID,),
        in_specs=[
            pl.BlockSpec((_BN, F_IN), lambda i: (i, 0)),
            pl.BlockSpec((F_IN, 4 * HID), lambda i: (0, 0)),
        ],
        out_specs=[blk, blk, blk, blk],
        out_shape=outs,
    )(xp, wcat)


def _tc_mid_body(m1_ref, y0_ref, s_ref, b1_ref, w_ref, z0_ref, u3_ref,
                 sz2_ref, sz1_ref):
    sv = s_ref[...]
    h = jnp.maximum(y0_ref[...] + sv * m1_ref[...] + b1_ref[...], 0.0)
    z = jnp.dot(h, w_ref[...], preferred_element_type=jnp.float32)
    z0_ref[...] = z[:, :NCLS]
    sz1_ref[...] = sv * z[:, NCLS:2 * NCLS]
    sz2_ref[...] = sv * z[:, 2 * NCLS:3 * NCLS]
    u3_ref[...] = sv * z[:, 3 * NCLS:]


def _tc_mid(m1, y0, svec, b1, wcat):
    f32 = jnp.float32
    outs = tuple(jax.ShapeDtypeStruct((NPAD, NCLS), f32) for _ in range(4))
    blk = pl.BlockSpec((_BN, NCLS), lambda i: (i, 0))
    return pl.pallas_call(
        _tc_mid_body,
        grid=(_GRID,),
        in_specs=[
            pl.BlockSpec((_BN, HID), lambda i: (i, 0)),
            pl.BlockSpec((_BN, HID), lambda i: (i, 0)),
            pl.BlockSpec((_BN, 1), lambda i: (i, 0)),
            pl.BlockSpec((1, HID), lambda i: (0, 0)),
            pl.BlockSpec((HID, 4 * NCLS), lambda i: (0, 0)),
        ],
        out_specs=[blk, blk, blk, blk],
        out_shape=outs,
    )(m1, y0, svec, b1, wcat)


def _tc_post_body(z0_ref, m2_ref, s_ref, b2_ref, out_ref):
    o = z0_ref[...] + s_ref[...] * m2_ref[...] + b2_ref[...]
    mx = jnp.max(o, axis=1, keepdims=True)
    ex = jnp.exp(o - mx)
    lse = mx + jnp.log(jnp.sum(ex, axis=1, keepdims=True))
    out_ref[...] = o - lse


def _tc_post(z0, m2, svec, b2):
    return pl.pallas_call(
        _tc_post_body,
        grid=(_GRID,),
        in_specs=[
            pl.BlockSpec((_BN, NCLS), lambda i: (i, 0)),
            pl.BlockSpec((_BN, NCLS), lambda i: (i, 0)),
            pl.BlockSpec((_BN, 1), lambda i: (i, 0)),
            pl.BlockSpec((1, NCLS), lambda i: (0, 0)),
        ],
        out_specs=pl.BlockSpec((_BN, NCLS), lambda i: (i, 0)),
        out_shape=jax.ShapeDtypeStruct((NPAD, NCLS), jnp.float32),
    )(z0, m2, svec, b2)


def _split_cols(a, w):
    return a.reshape(NPAD, 2, w).transpose(1, 0, 2).reshape(2 * NPAD, w)


def _merge_cols(a, w):
    return a.reshape(2, NPAD, w).transpose(1, 0, 2).reshape(NPAD, 2 * w)


def _fold(W):
    return jnp.concatenate(
        [W[0] - W[2], 3.0 * W[3] - W[1], 2.0 * W[2], -4.0 * W[3]], axis=1)


def kernel(x, edge_index, W1, b1, W2, b2):
    x = _f32(x)
    wcat1 = _fold(_f32(W1))
    wcat2 = _fold(_f32(W2))

    row = edge_index[0].astype(jnp.int32)
    col = edge_index[1].astype(jnp.int32)
    padv = (jnp.arange(E, E2, dtype=jnp.int32)) % N
    rowp = jnp.concatenate([row, padv]).reshape(E2 // 128, 128)
    colp_in = jnp.concatenate([col, padv]).reshape(E2 // 128, 128)
    rowoff = jnp.stack([rowp, rowp + NPAD])

    rowoff256 = rowoff.reshape(NC, E2 // 256, 256)

    degflat, colp = _edge_prep(rowp, colp_in)
    colp256 = colp.reshape(E2 // 256, 256)

    xp = jnp.pad(x, ((0, NPAD - N), (0, 0)))
    y0, y1, y2, y3 = _tc_prep(xp, wcat1)

    m1s, _, svec, s2flat = _prop64(rowoff, colp, _split_cols(y3, 64),
                                   _split_cols(y2, 64), _split_cols(y1, 64),
                                   degflat)
    m1 = _merge_cols(m1s, 64)
    svec2 = svec.reshape(NPAD, 1)

    z0, u3z, sz2, sz1 = _tc_mid(m1, y0, svec2, b1.reshape(1, HID), wcat2)

    m2s, _ = _prop32(rowoff256, colp256, _split_cols(u3z, 32),
                     _split_cols(sz2, 32), _split_cols(sz1, 32), s2flat)
    m2 = _merge_cols(m2s, 32)

    out = _tc_post(z0, m2, svec2, b2.reshape(1, NCLS))
    return out[:N]


# speedup vs baseline: 1.1129x; 1.0251x over previous
import functools

import jax
import jax.numpy as jnp
from jax import lax
from jax.experimental import pallas as pl
from jax.experimental.pallas import tpu as pltpu
from jax.experimental.pallas import tpu_sc as plsc

N = 10000
E = 320000
F_IN = 128
HID = 128
NCLS = 64

NC = 2
NS = 16
LANES = 16

NPAD = 10240
TRASHN = NPAD - N
E2 = 327680
EPT = E2 // NS
ECH = 128
NCHUNK = EPT // ECH
GB = 2
NPT = NPAD // NS
NODE_CH = 64
NNCH = NPT // NODE_CH

EPW = E2 // (NC * NS)
PCH = 2048
PROWS = PCH // 128

_mesh = plsc.VectorSubcoreMesh(core_axis_name="c", subcore_axis_name="s")


def _f32(x):
    return jnp.asarray(x, jnp.float32)


@functools.partial(
    pl.kernel,
    out_type=(
        jax.ShapeDtypeStruct((NC * NPAD,), jnp.float32),
        jax.ShapeDtypeStruct((E2 // 128, 128), jnp.int32),
    ),
    mesh=_mesh,
    scratch_types=dict(
        rbuf=pltpu.VMEM((PROWS, 128), jnp.int32),
        cbuf=pltpu.VMEM((PROWS, 128), jnp.int32),
        cpbuf=pltpu.VMEM((PROWS, 128), jnp.int32),
        wbuf=pltpu.VMEM((PROWS, 128), jnp.float32),
        zbuf=pltpu.VMEM((NPT,), jnp.float32),
        degacc=pltpu.VMEM_SHARED((NPAD,), jnp.float32),
        sem=pltpu.SemaphoreType.DMA,
    ),
)
def _edge_prep(row_hbm, col_hbm, deg_hbm, colp_hbm, rbuf, cbuf, cpbuf, wbuf,
               zbuf, degacc, sem):
    c = lax.axis_index("c")
    s = lax.axis_index("s")
    wid = c * NS + s

    def _z(t, _):
        zbuf[pl.ds(t * LANES, LANES)] = jnp.zeros((LANES,), jnp.float32)
        return _
    lax.fori_loop(0, NPT // LANES, _z, None)
    pltpu.sync_copy(zbuf, degacc.at[pl.ds(s * NPT, NPT)])
    plsc.subcore_barrier()

    def chunk(ch, _):
        base = wid * (EPW // 128) + ch * PROWS
        pltpu.sync_copy(row_hbm.at[pl.ds(base, PROWS)], rbuf)
        pltpu.sync_copy(col_hbm.at[pl.ds(base, PROWS)], cbuf)

        def vec(t, _):
            j = t // (128 // LANES)
            k = t % (128 // LANES)
            r = rbuf[j, pl.ds(k * LANES, LANES)]
            cc = cbuf[j, pl.ds(k * LANES, LANES)]
            m = r != cc
            cpbuf[j, pl.ds(k * LANES, LANES)] = jnp.where(
                m, cc, N + jnp.remainder(cc, TRASHN))
            wbuf[j, pl.ds(k * LANES, LANES)] = jnp.where(m, 1.0, 0.0)
            return _
        lax.fori_loop(0, PROWS * (128 // LANES), vec, None)

        pltpu.sync_copy(cpbuf, colp_hbm.at[pl.ds(base, PROWS)])
        for j in range(PROWS):
            pltpu.sync_copy(wbuf.at[j], degacc.at[rbuf.at[j]], add=True)
        return _
    lax.fori_loop(0, EPW // PCH, chunk, None)
    plsc.subcore_barrier()

    pltpu.sync_copy(degacc.at[pl.ds(s * NPT, NPT)], zbuf)
    pltpu.sync_copy(zbuf, deg_hbm.at[pl.ds(c * NPAD + s * NPT, NPT)])


def _make_prop(W, fuse_deg):
    QN = W // LANES
    CHW = 8192 // W
    NGW = EPT // CHW
    GROUPS = NGW // GB

    outs = (
        jax.ShapeDtypeStruct((NC * NPAD, W), jnp.float32),
        jax.ShapeDtypeStruct((NC * NPAD, W), jnp.float32),
    )
    if fuse_deg:
        outs = outs + (
            jax.ShapeDtypeStruct((NPAD,), jnp.float32),
            jax.ShapeDtypeStruct((NPAD,), jnp.float32),
        )

    scratch = dict(
        rixb=pltpu.VMEM((4, GB, CHW), jnp.int32),
        cixb=pltpu.VMEM((4, GB, CHW), jnp.int32),
        gbuf=pltpu.VMEM((2 * GB, CHW, W), jnp.float32),
        nsy=pltpu.VMEM((NODE_CH, W), jnp.float32),
        nu=pltpu.VMEM((NODE_CH, W), jnp.float32),
        zbuf=pltpu.VMEM((NODE_CH, W), jnp.float32),
        s2b=pltpu.VMEM((NPT,), jnp.float32),
        sb=pltpu.VMEM((NPT,), jnp.float32),
        acc_sh=pltpu.VMEM_SHARED((NPAD, W), jnp.float32),
        gsem=pltpu.SemaphoreType.DMA((2 * GB,)),
        ssem=pltpu.SemaphoreType.DMA((2 * GB,)),
        risem=pltpu.SemaphoreType.DMA((4,)),
        cisem=pltpu.SemaphoreType.DMA((4,)),
        asem=pltpu.SemaphoreType.DMA,
        bsem=pltpu.SemaphoreType.DMA,
    )
    @functools.partial(
        pl.kernel,
        out_type=outs,
        mesh=_mesh,
        scratch_types=scratch,
        compiler_params=pltpu.CompilerParams(needs_layout_passes=False,
                                             use_tc_tiling_on_sc=False),
    )
    def prop(rowoff_hbm, colx_hbm, u3_hbm, sy2_hbm, sy1_hbm, *rest,
             rixb, cixb, gbuf, nsy, nu, zbuf, s2b, sb, acc_sh, gsem, ssem,
             risem, cisem, asem, bsem, **xscr):
        del xscr
        if fuse_deg:
            deg_hbm, m_hbm, uw_hbm, sv_hbm, s2v_hbm = rest
        else:
            s2_hbm, m_hbm, uw_hbm = rest
        colp_hbm = colx_hbm
        c = lax.axis_index("c")
        s = lax.axis_index("s")
        nbase = s * NPT
        ebase = s * NGW
        row_hbm = rowoff_hbm.at[c]

        if fuse_deg:
            pltpu.async_copy(deg_hbm.at[pl.ds(nbase, NPT)], s2b, asem)
            pltpu.async_copy(deg_hbm.at[pl.ds(NPAD + nbase, NPT)], sb, bsem)
            pltpu.make_async_copy(deg_hbm.at[pl.ds(nbase, NPT)], s2b,
                                  asem).wait()
            pltpu.make_async_copy(deg_hbm.at[pl.ds(NPAD + nbase, NPT)], sb,
                                  bsem).wait()

            def _deg(t, _):
                sl = pl.ds(t * LANES, LANES)
                d = s2b[sl] + sb[sl]
                pos = d > 0.0
                i = plsc.bitcast(d, jnp.int32)
                i = jnp.full((LANES,), 0x5F3759DF, jnp.int32) - \
                    lax.shift_right_logical(i, jnp.full((LANES,), 1,
                                                        jnp.int32))
                y = plsc.bitcast(i, jnp.float32)
                for _it in range(3):
                    y = y * (1.5 - 0.5 * d * y * y)
                s2b[sl] = jnp.where(pos, 1.0 / d, 0.0)
                sb[sl] = jnp.where(pos, y, 0.0)
                return _
            lax.fori_loop(0, NPT // LANES, _deg, None)

            @pl.when(c == 0)
            def _():
                pltpu.sync_copy(sb, sv_hbm.at[pl.ds(nbase, NPT)])
                pltpu.sync_copy(s2b, s2v_hbm.at[pl.ds(nbase, NPT)])
        else:
            pltpu.sync_copy(s2_hbm.at[pl.ds(nbase, NPT)], s2b)

        def _z(t, _):
            def _zrow(q, _2):
                zbuf[t, pl.ds(q * LANES, LANES)] = jnp.zeros((LANES,),
                                                             jnp.float32)
                return _2
            lax.fori_loop(0, QN, _zrow, None)
            return _
        lax.fori_loop(0, NODE_CH, _z, None)

        for q in range(NNCH):
            r0 = nbase + q * NODE_CH
            pltpu.sync_copy(zbuf, acc_sh.at[pl.ds(r0, NODE_CH)])
            if fuse_deg:
                pltpu.sync_copy(u3_hbm.at[pl.ds(c * NPAD + r0, NODE_CH)],
                                nsy)

                def _su3(j, _):
                    jj = q * NODE_CH + j
                    sv = plsc.load_gather(
                        sb, [jnp.full((LANES,), jj, jnp.int32)])
                    for qq in range(QN):
                        sl = pl.ds(qq * LANES, LANES)
                        nu[j, sl] = sv * nsy[j, sl]
                    return _
                lax.fori_loop(0, NODE_CH, _su3, None)
                pltpu.sync_copy(nu, uw_hbm.at[pl.ds(c * NPAD + r0, NODE_CH)])
        plsc.subcore_barrier()

        def idx_load(g, p):
            pltpu.async_copy(row_hbm.at[pl.ds(ebase + g * GB, GB)],
                             rixb.at[p], risem.at[p])
            pltpu.async_copy(colp_hbm.at[pl.ds(ebase + g * GB, GB)],
                             cixb.at[p], cisem.at[p])

        def sweep(u_hbm):
            idx_load(0, 0)
            idx_load(1, 1)

            def do_group(g, k):
                p2 = k % 2
                po = (k + 2) % 4
                rix = rixb.at[k]
                cix = cixb.at[k]
                pltpu.make_async_copy(
                    row_hbm.at[pl.ds(ebase + g * GB, GB)], rix,
                    risem.at[k]).wait()

                @pl.when(g >= 2)
                def _():
                    for b in range(GB):
                        pltpu.make_async_copy(
                            gbuf.at[p2 * GB + b],
                            acc_sh.at[cixb.at[po].at[b]],
                            ssem.at[p2 * GB + b]).wait()

                @pl.when(g + 2 < GROUPS)
                def _():
                    idx_load(g + 2, po)

                pltpu.make_async_copy(
                    colp_hbm.at[pl.ds(ebase + g * GB, GB)], cix,
                    cisem.at[k]).wait()
                for b in range(GB):
                    pltpu.async_copy(u_hbm.at[rix.at[b]],
                                     gbuf.at[p2 * GB + b],
                                     gsem.at[p2 * GB + b])
                for b in range(GB):
                    pltpu.make_async_copy(u_hbm.at[rix.at[b]],
                                          gbuf.at[p2 * GB + b],
                                          gsem.at[p2 * GB + b]).wait()
                    pltpu.async_copy(gbuf.at[p2 * GB + b],
                                     acc_sh.at[cix.at[b]],
                                     ssem.at[p2 * GB + b], add=True)

            def quad(it, _):
                for k in range(4):
                    do_group(4 * it + k, k)
                return _
            lax.fori_loop(0, GROUPS // 4, quad, None)
            for k in (2, 3):
                p2 = k % 2
                for b in range(GB):
                    pltpu.make_async_copy(
                        gbuf.at[p2 * GB + b],
                        acc_sh.at[cixb.at[k].at[b]],
                        ssem.at[p2 * GB + b]).wait()
            plsc.subcore_barrier()

        def rescale(sy_hbm):
            for q in range(NNCH):
                r0 = nbase + q * NODE_CH
                pltpu.async_copy(acc_sh.at[pl.ds(r0, NODE_CH)], nu, asem)
                pltpu.async_copy(sy_hbm.at[pl.ds(c * NPAD + r0, NODE_CH)],
                                 nsy, bsem)
                pltpu.make_async_copy(acc_sh.at[pl.ds(r0, NODE_CH)], nu,
                                      asem).wait()
                pltpu.make_async_copy(sy_hbm.at[pl.ds(c * NPAD + r0, NODE_CH)],
                                      nsy, bsem).wait()

                def node(j, _):
                    jj = q * NODE_CH + j
                    idxv = jnp.full((LANES,), jj, jnp.int32)
                    s2v = plsc.load_gather(s2b, [idxv])
                    if fuse_deg:
                        sv = plsc.load_gather(sb, [idxv])
                    for qq in range(QN):
                        sl = pl.ds(qq * LANES, LANES)
                        if fuse_deg:
                            nu[j, sl] = sv * nsy[j, sl] + s2v * nu[j, sl]
                        else:
                            nu[j, sl] = nsy[j, sl] + s2v * nu[j, sl]
                    return _
                lax.fori_loop(0, NODE_CH, node, None)

                pltpu.sync_copy(nu, uw_hbm.at[pl.ds(c * NPAD + r0, NODE_CH)])
                pltpu.sync_copy(zbuf, acc_sh.at[pl.ds(r0, NODE_CH)])
            plsc.subcore_barrier()

        sweep(uw_hbm if fuse_deg else u3_hbm)
        rescale(sy2_hbm)
        sweep(uw_hbm)
        rescale(sy1_hbm)
        sweep(uw_hbm)

        for q in range(NNCH):
            r0 = nbase + q * NODE_CH
            pltpu.sync_copy(acc_sh.at[pl.ds(r0, NODE_CH)], nu)
            pltpu.sync_copy(nu, m_hbm.at[pl.ds(c * NPAD + r0, NODE_CH)])

    return prop


_prop64 = _make_prop(64, fuse_deg=True)
_prop32 = _make_prop(32, fuse_deg=False)


_BN = 128
_GRID = NPAD // _BN


def _tc_prep_body(x_ref, w_ref, y0_ref, y1_ref, y2_ref, y3_ref):
    y = jnp.dot(x_ref[...], w_ref[...], preferred_element_type=jnp.float32)
    y0_ref[...] = y[:, :HID]
    y1_ref[...] = y[:, HID:2 * HID]
    y2_ref[...] = y[:, 2 * HID:3 * HID]
    y3_ref[...] = y[:, 3 * HID:]


def _tc_prep(xp, wcat):
    f32 = jnp.float32
    outs = tuple(jax.ShapeDtypeStruct((NPAD, HID), f32) for _ in range(4))
    blk = pl.BlockSpec((_BN, HID), lambda i: (i, 0))
    return pl.pallas_call(
        _tc_prep_body,
        grid=(_GRID,),
        in_specs=[
            pl.BlockSpec((_BN, F_IN), lambda i: (i, 0)),
            pl.BlockSpec((F_IN, 4 * HID), lambda i: (0, 0)),
        ],
        out_specs=[blk, blk, blk, blk],
        out_shape=outs,
    )(xp, wcat)


def _tc_mid_body(m1_ref, y0_ref, s_ref, b1_ref, w_ref, z0_ref, u3_ref,
                 sz2_ref, sz1_ref):
    sv = s_ref[...]
    h = jnp.maximum(y0_ref[...] + sv * m1_ref[...] + b1_ref[...], 0.0)
    z = jnp.dot(h, w_ref[...], preferred_element_type=jnp.float32)
    z0_ref[...] = z[:, :NCLS]
    sz1_ref[...] = sv * z[:, NCLS:2 * NCLS]
    sz2_ref[...] = sv * z[:, 2 * NCLS:3 * NCLS]
    u3_ref[...] = sv * z[:, 3 * NCLS:]


def _tc_mid(m1, y0, svec, b1, wcat):
    f32 = jnp.float32
    outs = tuple(jax.ShapeDtypeStruct((NPAD, NCLS), f32) for _ in range(4))
    blk = pl.BlockSpec((_BN, NCLS), lambda i: (i, 0))
    return pl.pallas_call(
        _tc_mid_body,
        grid=(_GRID,),
        in_specs=[
            pl.BlockSpec((_BN, HID), lambda i: (i, 0)),
            pl.BlockSpec((_BN, HID), lambda i: (i, 0)),
            pl.BlockSpec((_BN, 1), lambda i: (i, 0)),
            pl.BlockSpec((1, HID), lambda i: (0, 0)),
            pl.BlockSpec((HID, 4 * NCLS), lambda i: (0, 0)),
        ],
        out_specs=[blk, blk, blk, blk],
        out_shape=outs,
    )(m1, y0, svec, b1, wcat)


def _tc_post_body(z0_ref, m2_ref, s_ref, b2_ref, out_ref):
    o = z0_ref[...] + s_ref[...] * m2_ref[...] + b2_ref[...]
    mx = jnp.max(o, axis=1, keepdims=True)
    ex = jnp.exp(o - mx)
    lse = mx + jnp.log(jnp.sum(ex, axis=1, keepdims=True))
    out_ref[...] = o - lse


def _tc_post(z0, m2, svec, b2):
    return pl.pallas_call(
        _tc_post_body,
        grid=(_GRID,),
        in_specs=[
            pl.BlockSpec((_BN, NCLS), lambda i: (i, 0)),
            pl.BlockSpec((_BN, NCLS), lambda i: (i, 0)),
            pl.BlockSpec((_BN, 1), lambda i: (i, 0)),
            pl.BlockSpec((1, NCLS), lambda i: (0, 0)),
        ],
        out_specs=pl.BlockSpec((_BN, NCLS), lambda i: (i, 0)),
        out_shape=jax.ShapeDtypeStruct((NPAD, NCLS), jnp.float32),
    )(z0, m2, svec, b2)


def _split_cols(a, w):
    return a.reshape(NPAD, 2, w).transpose(1, 0, 2).reshape(2 * NPAD, w)


def _merge_cols(a, w):
    return a.reshape(2, NPAD, w).transpose(1, 0, 2).reshape(NPAD, 2 * w)


def _fold(W):
    return jnp.concatenate(
        [W[0] - W[2], 3.0 * W[3] - W[1], 2.0 * W[2], -4.0 * W[3]], axis=1)


def kernel(x, edge_index, W1, b1, W2, b2):
    x = _f32(x)
    wcat1 = _fold(_f32(W1))
    wcat2 = _fold(_f32(W2))

    row = edge_index[0].astype(jnp.int32)
    col = edge_index[1].astype(jnp.int32)
    padv = (jnp.arange(E, E2, dtype=jnp.int32)) % N
    rowp = jnp.concatenate([row, padv]).reshape(E2 // 128, 128)
    colp_in = jnp.concatenate([col, padv]).reshape(E2 // 128, 128)
    rowoff = jnp.stack([rowp, rowp + NPAD])

    rowoff256 = rowoff.reshape(NC, E2 // 256, 256)

    degflat, colp = _edge_prep(rowp, colp_in)
    colp256 = colp.reshape(E2 // 256, 256)

    xp = jnp.pad(x, ((0, NPAD - N), (0, 0)))
    y0, y1, y2, y3 = _tc_prep(xp, wcat1)

    m1s, _, svec, s2flat = _prop64(rowoff, colp, _split_cols(y3, 64),
                                   _split_cols(y2, 64), _split_cols(y1, 64),
                                   degflat)
    m1 = _merge_cols(m1s, 64)
    svec2 = svec.reshape(NPAD, 1)

    z0, u3z, sz2, sz1 = _tc_mid(m1, y0, svec2, b1.reshape(1, HID), wcat2)

    m2s, _ = _prop32(rowoff256, colp256, _split_cols(u3z, 32),
                     _split_cols(sz2, 32), _split_cols(sz1, 32), s2flat)
    m2 = _merge_cols(m2s, 32)

    out = _tc_post(z0, m2, svec2, b2.reshape(1, NCLS))
    return out[:N]
